# Initial kernel scaffold; baseline (speedup 1.0000x reference)
import jax
import jax.numpy as jnp
from jax.experimental import pallas as pl


def kernel(x, edge_index, W1l, b1, W1r, W2l, b2, W2r):
    raise NotImplementedError("write your pallas kernel here")


# speedup vs baseline: 10.9224x; 10.9224x over previous
import jax
import jax.numpy as jnp
from jax import lax
from jax.experimental import pallas as pl
from jax.experimental.pallas import tpu as pltpu
from jax.experimental.pallas import tpu_sc as plsc

_NC = 2
_NS = 16
_W = 80


def _mm_bias_body(x_ref, w_ref, b_ref, o_ref):
    o_ref[...] = (
        jnp.dot(x_ref[...], w_ref[...], preferred_element_type=jnp.float32)
        + b_ref[...]
    )


def _mm_bias(x, w, b2d):
    n = x.shape[0]
    k = w.shape[1]
    return pl.pallas_call(
        _mm_bias_body,
        out_shape=jax.ShapeDtypeStruct((n, k), jnp.float32),
    )(x, w, b2d)


def _layer2_body(aggs_ref, xr_ref, w_ref, b_ref, o_ref):
    n = xr_ref.shape[0]
    h = jnp.maximum(aggs_ref[:n, :] + aggs_ref[n:, :] + xr_ref[...], 0.0)
    o_ref[...] = (
        jnp.dot(h, w_ref[...], preferred_element_type=jnp.float32) + b_ref[...]
    )


def _layer2(aggs, xr, w, b2d):
    n = xr.shape[0]
    k = w.shape[1]
    return pl.pallas_call(
        _layer2_body,
        out_shape=jax.ShapeDtypeStruct((n, k), jnp.float32),
    )(aggs, xr, w, b2d)


def _combine_body(aggs_ref, hr_ref, o_ref):
    n = hr_ref.shape[0]
    o_ref[...] = aggs_ref[:n, :] + aggs_ref[n:, :] + hr_ref[...]


def _combine(aggs, hr):
    return pl.pallas_call(
        _combine_body,
        out_shape=jax.ShapeDtypeStruct(hr.shape, jnp.float32),
    )(aggs, hr)


def _seg_sum_sc(feat, src2, dst2, zeros):
    n, f = feat.shape
    nw = _NC * _NS
    cpw = src2.shape[1]
    rpt = n // _NS
    assert cpw % 2 == 1 and src2.shape[0] == nw and n % _NS == 0

    mesh = plsc.VectorSubcoreMesh(
        core_axis_name="c", subcore_axis_name="s",
        num_cores=_NC, num_subcores=_NS)

    def body(feat_hbm, src_hbm, dst_hbm, zero_hbm, out_hbm,
             acc, sidx, didx, rb0, rb1, sem0, sem1):
        c = lax.axis_index("c")
        s = lax.axis_index("s")
        wid = s * _NC + c
        r0 = s * rpt
        pltpu.sync_copy(zero_hbm.at[s], acc.at[pl.ds(r0, rpt)])
        pltpu.sync_copy(src_hbm.at[wid], sidx)
        pltpu.sync_copy(dst_hbm.at[wid], didx)
        plsc.subcore_barrier()

        def gather(k, rb, sem):
            pltpu.async_copy(feat_hbm.at[sidx.at[k]], rb, sem)

        def gwait(rb, sem):
            pltpu.make_async_copy(feat_hbm.at[sidx.at[0]], rb, sem).wait()

        def scat(k, rb):
            pltpu.sync_copy(rb, acc.at[didx.at[k]], add=True)

        gather(0, rb0, sem0)
        gather(1, rb1, sem1)

        def loop_body(i, carry):
            k = i * 2
            gwait(rb0, sem0)
            scat(k, rb0)

            @pl.when(k + 2 < cpw)
            def _():
                gather(k + 2, rb0, sem0)

            gwait(rb1, sem1)
            scat(k + 1, rb1)

            @pl.when(k + 3 < cpw)
            def _():
                gather(k + 3, rb1, sem1)

            return carry

        lax.fori_loop(0, cpw // 2, loop_body, 0)
        gwait(rb0, sem0)
        scat(cpw - 1, rb0)

        plsc.subcore_barrier()
        pltpu.sync_copy(acc.at[pl.ds(r0, rpt)], out_hbm.at[c, s])

    kern = pl.kernel(
        body,
        out_type=jax.ShapeDtypeStruct((_NC, _NS, rpt, f), jnp.float32),
        mesh=mesh,
        scratch_types=[
            pltpu.VMEM_SHARED((n, f), jnp.float32),
            pltpu.VMEM((cpw, _W), jnp.int32),
            pltpu.VMEM((cpw, _W), jnp.int32),
            pltpu.VMEM((_W, f), jnp.float32),
            pltpu.VMEM((_W, f), jnp.float32),
            pltpu.SemaphoreType.DMA,
            pltpu.SemaphoreType.DMA,
        ],
        compiler_params=pltpu.CompilerParams(use_tc_tiling_on_sc=False),
    )
    return kern(feat, src2, dst2, zeros)


def kernel(x, edge_index, W1l, b1, W1r, W2l, b2, W2r):
    n, d = x.shape
    h = W1l.shape[0]
    c = W2l.shape[0]

    nw = _NC * _NS
    src2 = edge_index[0].reshape(nw, -1, _W)
    dst2 = edge_index[1].reshape(nw, -1, _W)
    rpt = n // _NS

    wt1 = jnp.concatenate([W1l, W1r], axis=0).T
    bias1 = jnp.concatenate([b1, jnp.zeros((h,), jnp.float32)])[None, :]
    y1 = _mm_bias(x, wt1, bias1)
    xl = y1[:, :h]
    xr = y1[:, h:]

    aggs1 = _seg_sum_sc(xl, src2, dst2, jnp.zeros((_NS, rpt, h), jnp.float32))
    aggs1 = aggs1.reshape(2 * n, h)

    wt2 = jnp.concatenate([W2l, W2r], axis=0).T
    bias2 = jnp.concatenate([b2, jnp.zeros((c,), jnp.float32)])[None, :]
    y2 = _layer2(aggs1, xr, wt2, bias2)
    hl = y2[:, :c]
    hr = y2[:, c:]

    aggs2 = _seg_sum_sc(hl, src2, dst2, jnp.zeros((_NS, rpt, c), jnp.float32))
    aggs2 = aggs2.reshape(2 * n, c)
    return _combine(aggs2, hr)


# speedup vs baseline: 13.3517x; 1.2224x over previous
import jax
import jax.numpy as jnp
from jax import lax
from jax.experimental import pallas as pl
from jax.experimental.pallas import tpu as pltpu
from jax.experimental.pallas import tpu_sc as plsc

_NC = 2
_NS = 16
_W = 80
_NB = 3


def _mm_split_body(x_ref, w_ref, b_ref, ol_ref, or_ref):
    h = ol_ref.shape[1]
    y = jnp.dot(x_ref[...], w_ref[...], preferred_element_type=jnp.float32)
    ol_ref[...] = y[:, :h]
    or_ref[...] = y[:, h:] + b_ref[...]


def _mm_split(x, w, b2d):
    n = x.shape[0]
    k = w.shape[1] // 2
    return pl.pallas_call(
        _mm_split_body,
        out_shape=[jax.ShapeDtypeStruct((n, k), jnp.float32),
                   jax.ShapeDtypeStruct((n, k), jnp.float32)],
    )(x, w, b2d)


def _layer2_body(aggs_ref, xr_ref, w_ref, b_ref, ol_ref, or_ref):
    n = xr_ref.shape[0]
    c = ol_ref.shape[1]
    h = jnp.maximum(aggs_ref[:n, :] + aggs_ref[n:, :] + xr_ref[...], 0.0)
    y = jnp.dot(h, w_ref[...], preferred_element_type=jnp.float32)
    ol_ref[...] = y[:, :c]
    or_ref[...] = y[:, c:] + b_ref[...]


def _layer2(aggs, xr, w, b2d):
    n = xr.shape[0]
    c = w.shape[1] // 2
    return pl.pallas_call(
        _layer2_body,
        out_shape=[jax.ShapeDtypeStruct((n, c), jnp.float32),
                   jax.ShapeDtypeStruct((n, c), jnp.float32)],
    )(aggs, xr, w, b2d)


def _combine_body(aggs_ref, hr_ref, o_ref):
    n = hr_ref.shape[0]
    o_ref[...] = aggs_ref[:n, :] + aggs_ref[n:, :] + hr_ref[...]


def _combine(aggs, hr):
    return pl.pallas_call(
        _combine_body,
        out_shape=jax.ShapeDtypeStruct(hr.shape, jnp.float32),
    )(aggs, hr)


def _seg_sum_sc(feat, src2, dst2):
    n, f = feat.shape
    nw = _NC * _NS
    cpw = src2.shape[1]
    rpt = n // _NS
    assert src2.shape[0] == nw and n % _NS == 0 and cpw % _NB == 2

    mesh = plsc.VectorSubcoreMesh(
        core_axis_name="c", subcore_axis_name="s",
        num_cores=_NC, num_subcores=_NS)

    def body(feat_hbm, src_hbm, dst_hbm, out_hbm,
             acc, sidx, didx, rbufs, gsems, ssems):
        c = lax.axis_index("c")
        s = lax.axis_index("s")
        wid = s * _NC + c
        r0 = s * rpt

        pltpu.async_copy(src_hbm.at[wid], sidx, gsems[0])
        pltpu.async_copy(dst_hbm.at[wid], didx, gsems[1])

        zv = jnp.zeros((16,), jnp.float32)

        def zb(r, carry):
            for q in range(f // 16):
                rbufs[0][r, pl.ds(q * 16, 16)] = zv
            return carry

        lax.fori_loop(0, _W, zb, 0)
        nfull = rpt // _W
        for t in range(nfull):
            pltpu.sync_copy(rbufs[0], acc.at[pl.ds(r0 + t * _W, _W)])
        rem = rpt - nfull * _W
        if rem:
            pltpu.sync_copy(rbufs[0].at[pl.ds(0, rem)],
                            acc.at[pl.ds(r0 + nfull * _W, rem)])

        pltpu.make_async_copy(src_hbm.at[wid], sidx, gsems[0]).wait()
        pltpu.make_async_copy(dst_hbm.at[wid], didx, gsems[1]).wait()

        def gather(k, b):
            pltpu.async_copy(feat_hbm.at[sidx.at[k]], rbufs[b], gsems[b])

        def gwait(b):
            pltpu.make_async_copy(feat_hbm.at[sidx.at[0]], rbufs[b],
                                  gsems[b]).wait()

        def scat(k, b):
            pltpu.async_copy(rbufs[b], acc.at[didx.at[k]], ssems[b],
                             add=True)

        def swait(b):
            pltpu.make_async_copy(rbufs[b], acc.at[didx.at[0]],
                                  ssems[b]).wait()

        gather(0, 0)
        gather(1, 1)
        plsc.subcore_barrier()

        def loop_body(i, carry):
            for b in range(_NB):
                k = i * _NB + b
                gwait(b)
                scat(k, b)
                b2 = (b + 2) % _NB

                @pl.when(k + 2 < cpw)
                def _():
                    @pl.when(k >= 1)
                    def _():
                        swait(b2)

                    gather(k + 2, b2)
            return carry

        lax.fori_loop(0, (cpw - 2) // _NB, loop_body, 0)
        for k in range(cpw - 2, cpw):
            b = k % _NB
            gwait(b)
            scat(k, b)
        for b in range(_NB):
            swait(b)

        plsc.subcore_barrier()
        pltpu.sync_copy(acc.at[pl.ds(r0, rpt)], out_hbm.at[c, s])

    kern = pl.kernel(
        body,
        out_type=jax.ShapeDtypeStruct((_NC, _NS, rpt, f), jnp.float32),
        mesh=mesh,
        scratch_types=[
            pltpu.VMEM_SHARED((n, f), jnp.float32),
            pltpu.VMEM((cpw, _W), jnp.int32),
            pltpu.VMEM((cpw, _W), jnp.int32),
            [pltpu.VMEM((_W, f), jnp.float32) for _ in range(_NB)],
            [pltpu.SemaphoreType.DMA for _ in range(_NB)],
            [pltpu.SemaphoreType.DMA for _ in range(_NB)],
        ],
        compiler_params=pltpu.CompilerParams(use_tc_tiling_on_sc=False),
    )
    return kern(feat, src2, dst2)


def kernel(x, edge_index, W1l, b1, W1r, W2l, b2, W2r):
    n, d = x.shape
    h = W1l.shape[0]
    c = W2l.shape[0]

    nw = _NC * _NS
    src2 = edge_index[0].reshape(nw, -1, _W)
    dst2 = edge_index[1].reshape(nw, -1, _W)

    wt1 = jnp.concatenate([W1l, W1r], axis=0).T
    xl, xr = _mm_split(x, wt1, b1[None, :])

    aggs1 = _seg_sum_sc(xl, src2, dst2).reshape(2 * n, h)

    wt2 = jnp.concatenate([W2l, W2r], axis=0).T
    hl, hr = _layer2(aggs1, xr, wt2, b2[None, :])

    aggs2 = _seg_sum_sc(hl, src2, dst2).reshape(2 * n, c)
    return _combine(aggs2, hr)
